# baseline (speedup 1.0000x reference)
import functools

import jax
import jax.numpy as jnp
from jax.experimental import pallas as pl

N = 50000
E = 800000
DC = 4
DL = 10
H = 16
NL = 5
EC = 32
DD = DL * DC
NH = NL * H

ET = 3200
EG = E // ET


def _edge1_body(ef_ref, g1_ref, wde_ref, bdn_ref, m1_ref):
    ef = ef_ref[...]
    e1 = jnp.dot(ef, wde_ref[...], preferred_element_type=jnp.float32)
    m1_ref[...] = jnp.maximum(g1_ref[...] + e1 + bdn_ref[...], 0.0)


def _edge2_body(ef_ref, ga_ref, gb_ref, gp_ref, w1e_ref, b1_ref, w2_ref,
                b2_ref, lng_ref, lnb_ref, wnde_ref, bnd_ref, m3_ref):
    ef = ef_ref[...]
    e2 = jnp.dot(ef, w1e_ref[...], preferred_element_type=jnp.float32)
    h1 = jnp.maximum(ga_ref[...] + gb_ref[...] + e2 + b1_ref[...], 0.0)
    upd = jnp.dot(h1, w2_ref[...], preferred_element_type=jnp.float32) + b2_ref[...]
    mu = jnp.mean(upd, axis=-1, keepdims=True)
    var = jnp.mean((upd - mu) ** 2, axis=-1, keepdims=True)
    ln = (upd - mu) * jax.lax.rsqrt(var + 1e-5) * lng_ref[...] + lnb_ref[...]
    edge_out = ef + ln
    e3 = jnp.dot(edge_out, wnde_ref[...], preferred_element_type=jnp.float32)
    m3_ref[...] = jnp.maximum(gp_ref[...] + e3 + bnd_ref[...], 0.0)


def _edge_spec(width):
    return pl.BlockSpec((ET, width), lambda i: (i, 0))


def _w_spec(shape):
    return pl.BlockSpec(shape, lambda i: tuple(0 for _ in shape))


def kernel(density_features, node_features, edge_features, edge_index,
           W_dn, b_dn, W1, b1, W2, b2, ln_g, ln_b, W_nd, b_nd):
    src = edge_index[0]
    dst = edge_index[1]

    d_flat = density_features.reshape(N, DD)
    deg = jax.ops.segment_sum(jnp.ones((E,), jnp.float32), dst, num_segments=N)

    P_d = d_flat @ W_dn[:DD]
    G1 = jnp.take(P_d, src, axis=0)

    m1 = pl.pallas_call(
        _edge1_body,
        grid=(EG,),
        in_specs=[_edge_spec(EC), _edge_spec(NH), _w_spec((EC, NH)),
                  _w_spec((NH,))],
        out_specs=_edge_spec(NH),
        out_shape=jax.ShapeDtypeStruct((E, NH), jnp.float32),
    )(edge_features, G1, W_dn[DD:], b_dn)

    inv_deg = 1.0 / (deg[:, None] + 1e-6)
    node_upd = jax.ops.segment_sum(m1, dst, num_segments=N) * inv_deg
    node_h = node_features + node_upd.reshape(N, NL, H)
    scale = jax.lax.rsqrt(jnp.mean(node_h * node_h, axis=(1, 2), keepdims=True) + 1e-6)
    node_h = node_h * scale

    node_inv = jnp.concatenate([node_h[:, 0, :], node_h[:, 4, :]], axis=-1)
    A = node_inv @ W1[:EC]
    B = node_inv @ W1[EC:2 * EC]
    n_flat = node_h.reshape(N, NH)
    P_n = n_flat @ W_nd[:NH]

    GA = jnp.take(A, src, axis=0)
    GB = jnp.take(B, dst, axis=0)
    GP = jnp.take(P_n, src, axis=0)

    m3 = pl.pallas_call(
        _edge2_body,
        grid=(EG,),
        in_specs=[_edge_spec(EC), _edge_spec(EC), _edge_spec(EC),
                  _edge_spec(DD), _w_spec((EC, EC)), _w_spec((EC,)),
                  _w_spec((EC, EC)), _w_spec((EC,)), _w_spec((EC,)),
                  _w_spec((EC,)), _w_spec((EC, DD)), _w_spec((DD,))],
        out_specs=_edge_spec(DD),
        out_shape=jax.ShapeDtypeStruct((E, DD), jnp.float32),
    )(edge_features, GA, GB, GP, W1[2 * EC:], b1, W2, b2, ln_g, ln_b,
      W_nd[NH:], b_nd)

    dens_upd = jax.ops.segment_sum(m3, dst, num_segments=N) * inv_deg
    dens = density_features + dens_upd.reshape(N, DL, DC)
    dscale = jax.lax.rsqrt(jnp.mean(dens * dens, axis=(1, 2), keepdims=True) + 1e-6)
    return dens * dscale


# speedup vs baseline: 1.8165x; 1.8165x over previous
import functools

import jax
import jax.numpy as jnp
from jax import lax
from jax.experimental import pallas as pl
from jax.experimental.pallas import tpu as pltpu
from jax.experimental.pallas import tpu_sc as plsc

N = 50000
E = 800000
DC = 4
DL = 10
H = 16
NL = 5
EC = 32
DD = DL * DC
NH = NL * H

ET = 3200
EG = E // ET

NSC = 2
NSS = 16
NWK = NSC * NSS
IW = 125
CH = 8
ER = E // IW
RPW = ER // NWK
CPW = RPW // CH


def _sc_mesh():
    return plsc.VectorSubcoreMesh(core_axis_name="c", subcore_axis_name="s",
                                  num_cores=NSC, num_subcores=NSS)


def _gather_rows(table_hbm, idx_hbm, out_hbm, idx_v, gbuf, sem):
    wid = lax.axis_index("s") * NSC + lax.axis_index("c")
    base = wid * RPW

    def chunk(i, _):
        r0 = base + i * CH
        pltpu.sync_copy(idx_hbm.at[pl.ds(r0, CH)], idx_v)
        cps = []
        for j in range(CH):
            cps.append(pltpu.async_copy(table_hbm.at[idx_v.at[j]],
                                        gbuf.at[j], sem))
        for cp in cps:
            cp.wait()
        pltpu.sync_copy(gbuf, out_hbm.at[pl.ds(r0, CH)])
        return ()

    lax.fori_loop(0, CPW, chunk, (), unroll=False)


def _sc_gather(table, idx2, width):
    f = pl.kernel(
        functools.partial(_gather_rows),
        out_type=jax.ShapeDtypeStruct((ER, IW, width), jnp.float32),
        mesh=_sc_mesh(),
        scratch_types=[
            pltpu.VMEM((CH, IW), jnp.int32),
            pltpu.VMEM((CH, IW, width), jnp.float32),
            pltpu.SemaphoreType.DMA,
        ],
        compiler_params=pltpu.CompilerParams(use_tc_tiling_on_sc=False),
    )
    return f(table, idx2).reshape(E, width)


NPT = N // NSS
RPT = ER // NSS
SCH = RPT // CH


def _seg_half(m_hbm, dst_hbm, zeros_hbm, out_hbm, idx_v, mbuf, acc):
    sid = lax.axis_index("s")
    pltpu.sync_copy(zeros_hbm, acc.at[pl.ds(sid * NPT, NPT)])
    plsc.subcore_barrier()

    def chunk(i, _):
        r0 = sid * RPT + i * CH
        pltpu.sync_copy(dst_hbm.at[pl.ds(r0, CH)], idx_v)
        pltpu.sync_copy(m_hbm.at[pl.ds(r0, CH)], mbuf)
        for j in range(CH):
            pltpu.sync_copy(mbuf.at[j], acc.at[idx_v.at[j]], add=True)
        return ()

    lax.fori_loop(0, SCH, chunk, (), unroll=False)
    plsc.subcore_barrier()
    pltpu.sync_copy(acc.at[pl.ds(sid * NPT, NPT)],
                    out_hbm.at[pl.ds(sid * NPT, NPT)])


def _make_seg_body(nparts):
    def body(*refs):
        ms = refs[:nparts]
        dst_hbm = refs[nparts]
        zeros_hbm = refs[nparts + 1]
        outs = refs[nparts + 2:nparts + 2 + nparts]
        idx_v, mbuf, acc = refs[-3:]
        cid = lax.axis_index("c")
        half = (nparts + 1) // 2

        @pl.when(cid == 0)
        def _():
            for q in range(half):
                _seg_half(ms[q], dst_hbm, zeros_hbm, outs[q], idx_v, mbuf, acc)

        @pl.when(cid == 1)
        def _():
            for q in range(half, nparts):
                _seg_half(ms[q], dst_hbm, zeros_hbm, outs[q], idx_v, mbuf, acc)

    return body


SW = 16


def _sc_segsum(parts, dst2):
    nparts = len(parts)
    zeros = jnp.zeros((NPT, SW), jnp.float32)
    f = pl.kernel(
        _make_seg_body(nparts),
        out_type=tuple(jax.ShapeDtypeStruct((N, SW), jnp.float32)
                       for _ in range(nparts)),
        mesh=_sc_mesh(),
        scratch_types=[
            pltpu.VMEM((CH, IW), jnp.int32),
            pltpu.VMEM((CH, IW, SW), jnp.float32),
            pltpu.VMEM_SHARED((N, SW), jnp.float32),
        ],
        compiler_params=pltpu.CompilerParams(use_tc_tiling_on_sc=False),
    )
    return f(*parts, dst2, zeros)


def _edge1_body(ef_ref, g1_ref, wde_ref, bdn_ref, *m_refs):
    ef = ef_ref[...]
    e1 = jnp.dot(ef, wde_ref[...], preferred_element_type=jnp.float32)
    m1 = jnp.maximum(g1_ref[...] + e1 + bdn_ref[...], 0.0)
    for q, r in enumerate(m_refs):
        r[...] = m1[:, q * SW:(q + 1) * SW]


def _edge2_body(ef_ref, ga_ref, gb_ref, gp_ref, w1e_ref, b1_ref, w2_ref,
                b2_ref, lng_ref, lnb_ref, wnde_ref, bnd_ref, m3_ref):
    ef = ef_ref[...]
    e2 = jnp.dot(ef, w1e_ref[...], preferred_element_type=jnp.float32)
    h1 = jnp.maximum(ga_ref[...] + gb_ref[...] + e2 + b1_ref[...], 0.0)
    upd = jnp.dot(h1, w2_ref[...], preferred_element_type=jnp.float32) + b2_ref[...]
    mu = jnp.mean(upd, axis=-1, keepdims=True)
    var = jnp.mean((upd - mu) ** 2, axis=-1, keepdims=True)
    ln = (upd - mu) * jax.lax.rsqrt(var + 1e-5) * lng_ref[...] + lnb_ref[...]
    edge_out = ef + ln
    e3 = jnp.dot(edge_out, wnde_ref[...], preferred_element_type=jnp.float32)
    m3 = jnp.maximum(gp_ref[...] + e3 + bnd_ref[...], 0.0)
    m3_ref[...] = m3


def _edge2_body_split(ef_ref, g2_ref, gb_ref, w1e_ref, b1_ref,
                      w2_ref, b2_ref, lng_ref, lnb_ref, wnde_ref, bnd_ref,
                      m3a_ref, m3b_ref, m3c_ref):
    ef = ef_ref[...]
    ga = g2_ref[:, :EC]
    gp = g2_ref[:, EC:EC + DD]
    e2 = jnp.dot(ef, w1e_ref[...], preferred_element_type=jnp.float32)
    h1 = jnp.maximum(ga + gb_ref[...] + e2 + b1_ref[...], 0.0)
    upd = jnp.dot(h1, w2_ref[...], preferred_element_type=jnp.float32) + b2_ref[...]
    mu = jnp.mean(upd, axis=-1, keepdims=True)
    var = jnp.mean((upd - mu) ** 2, axis=-1, keepdims=True)
    ln = (upd - mu) * jax.lax.rsqrt(var + 1e-5) * lng_ref[...] + lnb_ref[...]
    edge_out = ef + ln
    e3 = jnp.dot(edge_out, wnde_ref[...], preferred_element_type=jnp.float32)
    m3 = jnp.maximum(gp + e3 + bnd_ref[...], 0.0)
    m3a_ref[...] = m3[:, :SW]
    m3b_ref[...] = m3[:, SW:2 * SW]
    m3c_ref[...] = jnp.concatenate(
        [m3[:, 2 * SW:], jnp.zeros((m3.shape[0], 3 * SW - DD), jnp.float32)],
        axis=-1)


def _edge_spec(width):
    return pl.BlockSpec((ET, width), lambda i: (i, 0))


def _w_spec(shape):
    return pl.BlockSpec(shape, lambda i: tuple(0 for _ in shape))


def kernel(density_features, node_features, edge_features, edge_index,
           W_dn, b_dn, W1, b1, W2, b2, ln_g, ln_b, W_nd, b_nd):
    src = edge_index[0]
    dst = edge_index[1]

    d_flat = density_features.reshape(N, DD)
    deg = jax.ops.segment_sum(jnp.ones((E,), jnp.float32), dst, num_segments=N)

    src2 = src.reshape(ER, IW)
    dst2 = dst.reshape(ER, IW)

    P_d = d_flat @ W_dn[:DD]
    G1 = _sc_gather(P_d, src2, NH)

    m1q = pl.pallas_call(
        _edge1_body,
        grid=(EG,),
        in_specs=[_edge_spec(EC), _edge_spec(NH), _w_spec((EC, NH)),
                  _w_spec((NH,))],
        out_specs=tuple(_edge_spec(SW) for _ in range(NH // SW)),
        out_shape=tuple(jax.ShapeDtypeStruct((E, SW), jnp.float32)
                        for _ in range(NH // SW)),
    )(edge_features, G1, W_dn[DD:], b_dn)

    acc1 = _sc_segsum([m.reshape(ER, IW, SW) for m in m1q], dst2)

    inv_deg = 1.0 / (deg[:, None] + 1e-6)
    node_upd = jnp.concatenate(acc1, axis=-1) * inv_deg
    node_h = node_features + node_upd.reshape(N, NL, H)
    scale = jax.lax.rsqrt(jnp.mean(node_h * node_h, axis=(1, 2), keepdims=True) + 1e-6)
    node_h = node_h * scale

    node_inv = jnp.concatenate([node_h[:, 0, :], node_h[:, 4, :]], axis=-1)
    A = node_inv @ W1[:EC]
    B = node_inv @ W1[EC:2 * EC]
    n_flat = node_h.reshape(N, NH)
    P_n = n_flat @ W_nd[:NH]

    T2 = jnp.concatenate([A, P_n, jnp.zeros((N, NH - EC - DD), jnp.float32)],
                         axis=-1)
    G2 = _sc_gather(T2, src2, NH)
    GB = _sc_gather(B, dst2, EC)

    m3parts = pl.pallas_call(
        _edge2_body_split,
        grid=(EG,),
        in_specs=[_edge_spec(EC), _edge_spec(NH), _edge_spec(EC),
                  _w_spec((EC, EC)), _w_spec((EC,)),
                  _w_spec((EC, EC)), _w_spec((EC,)), _w_spec((EC,)),
                  _w_spec((EC,)), _w_spec((EC, DD)), _w_spec((DD,))],
        out_specs=tuple(_edge_spec(SW) for _ in range(3)),
        out_shape=tuple(jax.ShapeDtypeStruct((E, SW), jnp.float32)
                        for _ in range(3)),
    )(edge_features, G2, GB, W1[2 * EC:], b1, W2, b2, ln_g, ln_b,
      W_nd[NH:], b_nd)

    acc3 = _sc_segsum([m.reshape(ER, IW, SW) for m in m3parts], dst2)
    dens_upd = jnp.concatenate(acc3, axis=-1)[:, :DD] * inv_deg
    dens = density_features + dens_upd.reshape(N, DL, DC)
    dscale = jax.lax.rsqrt(jnp.mean(dens * dens, axis=(1, 2), keepdims=True) + 1e-6)
    return dens * dscale


# speedup vs baseline: 2.0614x; 1.1348x over previous
import functools

import jax
import jax.numpy as jnp
from jax import lax
from jax.experimental import pallas as pl
from jax.experimental.pallas import tpu as pltpu
from jax.experimental.pallas import tpu_sc as plsc

N = 50000
E = 800000
DC = 4
DL = 10
H = 16
NL = 5
EC = 32
DD = DL * DC
NH = NL * H

ET = 3200
EG = E // ET

NSC = 2
NSS = 16
NWK = NSC * NSS
IW = 125
CH = 8
ER = E // IW
RPW = ER // NWK
CPW = RPW // CH


def _sc_mesh():
    return plsc.VectorSubcoreMesh(core_axis_name="c", subcore_axis_name="s",
                                  num_cores=NSC, num_subcores=NSS)


def _gather_rows(table_hbm, idx_hbm, out_hbm, idx_v, gbuf, sem):
    wid = lax.axis_index("s") * NSC + lax.axis_index("c")
    base = wid * RPW

    def chunk(i, _):
        r0 = base + i * CH
        pltpu.sync_copy(idx_hbm.at[pl.ds(r0, CH)], idx_v)
        cps = []
        for j in range(CH):
            cps.append(pltpu.async_copy(table_hbm.at[idx_v.at[j]],
                                        gbuf.at[j], sem))
        for cp in cps:
            cp.wait()
        pltpu.sync_copy(gbuf, out_hbm.at[pl.ds(r0, CH)])
        return ()

    lax.fori_loop(0, CPW, chunk, (), unroll=False)


def _sc_gather(table, idx2, width):
    f = pl.kernel(
        functools.partial(_gather_rows),
        out_type=jax.ShapeDtypeStruct((ER, IW, width), jnp.float32),
        mesh=_sc_mesh(),
        scratch_types=[
            pltpu.VMEM((CH, IW), jnp.int32),
            pltpu.VMEM((CH, IW, width), jnp.float32),
            pltpu.SemaphoreType.DMA,
        ],
        compiler_params=pltpu.CompilerParams(use_tc_tiling_on_sc=False),
    )
    return f(table, idx2).reshape(E, width)


NPT = N // NSS
RPT = ER // NSS
SCH = RPT // CH


def _seg_half(m_hbm, dst_hbm, zeros_hbm, out_hbm, idx_v, mbuf, acc):
    sid = lax.axis_index("s")
    pltpu.sync_copy(zeros_hbm, acc.at[pl.ds(sid * NPT, NPT)])
    plsc.subcore_barrier()

    def chunk(i, _):
        r0 = sid * RPT + i * CH
        pltpu.sync_copy(dst_hbm.at[pl.ds(r0, CH)], idx_v)
        pltpu.sync_copy(m_hbm.at[pl.ds(r0, CH)], mbuf)
        for j in range(CH):
            pltpu.sync_copy(mbuf.at[j], acc.at[idx_v.at[j]], add=True)
        return ()

    lax.fori_loop(0, SCH, chunk, (), unroll=False)
    plsc.subcore_barrier()
    pltpu.sync_copy(acc.at[pl.ds(sid * NPT, NPT)],
                    out_hbm.at[pl.ds(sid * NPT, NPT)])


def _deg_round(dst_hbm, ones_hbm, zeros_hbm, out_hbm, idx_v, ones_v, acc):
    sid = lax.axis_index("s")
    pltpu.sync_copy(ones_hbm, ones_v)
    pltpu.sync_copy(zeros_hbm, acc.at[pl.ds(sid * NPT, NPT)])
    plsc.subcore_barrier()

    def chunk(i, _):
        r0 = sid * RPT + i * CH
        pltpu.sync_copy(dst_hbm.at[pl.ds(r0, CH)], idx_v)
        for j in range(CH):
            pltpu.sync_copy(ones_v, acc.at[idx_v.at[j]], add=True)
        return ()

    lax.fori_loop(0, SCH, chunk, (), unroll=False)
    plsc.subcore_barrier()
    pltpu.sync_copy(acc.at[pl.ds(sid * NPT, NPT)],
                    out_hbm.at[pl.ds(sid * NPT, NPT)])


def _make_seg_body(nparts, with_deg):
    def body(*refs):
        ms = refs[:nparts]
        dst_hbm = refs[nparts]
        zeros_hbm = refs[nparts + 1]
        k = nparts + 2
        ones_hbm = refs[k] if with_deg else None
        k += 1 if with_deg else 0
        outs = refs[k:k + nparts]
        deg_out = refs[k + nparts] if with_deg else None
        if with_deg:
            idx_v, mbuf, acc, ones_v = refs[-4:]
        else:
            idx_v, mbuf, acc = refs[-3:]
        cid = lax.axis_index("c")
        half = (nparts + 1) // 2

        @pl.when(cid == 0)
        def _():
            for q in range(half):
                _seg_half(ms[q], dst_hbm, zeros_hbm, outs[q], idx_v, mbuf, acc)

        @pl.when(cid == 1)
        def _():
            for q in range(half, nparts):
                _seg_half(ms[q], dst_hbm, zeros_hbm, outs[q], idx_v, mbuf, acc)
            if with_deg:
                _deg_round(dst_hbm, ones_hbm, zeros_hbm, deg_out,
                           idx_v, ones_v, acc)

    return body


SW = 16


def _sc_segsum(parts, dst2, with_deg=False):
    nparts = len(parts)
    zeros = jnp.zeros((NPT, SW), jnp.float32)
    nouts = nparts + (1 if with_deg else 0)
    scratch = [
        pltpu.VMEM((CH, IW), jnp.int32),
        pltpu.VMEM((CH, IW, SW), jnp.float32),
        pltpu.VMEM_SHARED((N, SW), jnp.float32),
    ]
    args = list(parts) + [dst2, zeros]
    if with_deg:
        args.append(jnp.ones((IW, SW), jnp.float32))
        scratch.append(pltpu.VMEM((IW, SW), jnp.float32))
    f = pl.kernel(
        _make_seg_body(nparts, with_deg),
        out_type=tuple(jax.ShapeDtypeStruct((N, SW), jnp.float32)
                       for _ in range(nouts)),
        mesh=_sc_mesh(),
        scratch_types=scratch,
        compiler_params=pltpu.CompilerParams(use_tc_tiling_on_sc=False),
    )
    return f(*args)


def _edge1_body(ef_ref, g1_ref, wde_ref, bdn_ref, *m_refs):
    ef = ef_ref[...]
    e1 = jnp.dot(ef, wde_ref[...], preferred_element_type=jnp.float32)
    m1 = jnp.maximum(g1_ref[...] + e1 + bdn_ref[...], 0.0)
    for q, r in enumerate(m_refs):
        r[...] = m1[:, q * SW:(q + 1) * SW]


def _edge2_body(ef_ref, ga_ref, gb_ref, gp_ref, w1e_ref, b1_ref, w2_ref,
                b2_ref, lng_ref, lnb_ref, wnde_ref, bnd_ref, m3_ref):
    ef = ef_ref[...]
    e2 = jnp.dot(ef, w1e_ref[...], preferred_element_type=jnp.float32)
    h1 = jnp.maximum(ga_ref[...] + gb_ref[...] + e2 + b1_ref[...], 0.0)
    upd = jnp.dot(h1, w2_ref[...], preferred_element_type=jnp.float32) + b2_ref[...]
    mu = jnp.mean(upd, axis=-1, keepdims=True)
    var = jnp.mean((upd - mu) ** 2, axis=-1, keepdims=True)
    ln = (upd - mu) * jax.lax.rsqrt(var + 1e-5) * lng_ref[...] + lnb_ref[...]
    edge_out = ef + ln
    e3 = jnp.dot(edge_out, wnde_ref[...], preferred_element_type=jnp.float32)
    m3 = jnp.maximum(gp_ref[...] + e3 + bnd_ref[...], 0.0)
    m3_ref[...] = m3


def _edge2_body_split(ef_ref, g2_ref, gb_ref, w1e_ref, b1_ref,
                      w2_ref, b2_ref, lng_ref, lnb_ref, wnde_ref, bnd_ref,
                      m3a_ref, m3b_ref, m3c_ref):
    ef = ef_ref[...]
    ga = g2_ref[:, :EC]
    gp = g2_ref[:, EC:EC + DD]
    e2 = jnp.dot(ef, w1e_ref[...], preferred_element_type=jnp.float32)
    h1 = jnp.maximum(ga + gb_ref[...] + e2 + b1_ref[...], 0.0)
    upd = jnp.dot(h1, w2_ref[...], preferred_element_type=jnp.float32) + b2_ref[...]
    mu = jnp.mean(upd, axis=-1, keepdims=True)
    var = jnp.mean((upd - mu) ** 2, axis=-1, keepdims=True)
    ln = (upd - mu) * jax.lax.rsqrt(var + 1e-5) * lng_ref[...] + lnb_ref[...]
    edge_out = ef + ln
    e3 = jnp.dot(edge_out, wnde_ref[...], preferred_element_type=jnp.float32)
    m3 = jnp.maximum(gp + e3 + bnd_ref[...], 0.0)
    m3a_ref[...] = m3[:, :SW]
    m3b_ref[...] = m3[:, SW:2 * SW]
    m3c_ref[...] = jnp.concatenate(
        [m3[:, 2 * SW:], jnp.zeros((m3.shape[0], 3 * SW - DD), jnp.float32)],
        axis=-1)


def _edge_spec(width):
    return pl.BlockSpec((ET, width), lambda i: (i, 0))


def _w_spec(shape):
    return pl.BlockSpec(shape, lambda i: tuple(0 for _ in shape))


def kernel(density_features, node_features, edge_features, edge_index,
           W_dn, b_dn, W1, b1, W2, b2, ln_g, ln_b, W_nd, b_nd):
    src = edge_index[0]
    dst = edge_index[1]

    d_flat = density_features.reshape(N, DD)

    src2 = src.reshape(ER, IW)
    dst2 = dst.reshape(ER, IW)

    P_d = d_flat @ W_dn[:DD]
    G1 = _sc_gather(P_d, src2, NH)

    m1q = pl.pallas_call(
        _edge1_body,
        grid=(EG,),
        in_specs=[_edge_spec(EC), _edge_spec(NH), _w_spec((EC, NH)),
                  _w_spec((NH,))],
        out_specs=tuple(_edge_spec(SW) for _ in range(NH // SW)),
        out_shape=tuple(jax.ShapeDtypeStruct((E, SW), jnp.float32)
                        for _ in range(NH // SW)),
    )(edge_features, G1, W_dn[DD:], b_dn)

    *acc1, degc = _sc_segsum([m.reshape(ER, IW, SW) for m in m1q], dst2,
                             with_deg=True)
    deg = degc[:, 0]

    inv_deg = 1.0 / (deg[:, None] + 1e-6)
    node_upd = jnp.concatenate(acc1, axis=-1) * inv_deg
    node_h = node_features + node_upd.reshape(N, NL, H)
    scale = jax.lax.rsqrt(jnp.mean(node_h * node_h, axis=(1, 2), keepdims=True) + 1e-6)
    node_h = node_h * scale

    node_inv = jnp.concatenate([node_h[:, 0, :], node_h[:, 4, :]], axis=-1)
    A = node_inv @ W1[:EC]
    B = node_inv @ W1[EC:2 * EC]
    n_flat = node_h.reshape(N, NH)
    P_n = n_flat @ W_nd[:NH]

    T2 = jnp.concatenate([A, P_n, jnp.zeros((N, NH - EC - DD), jnp.float32)],
                         axis=-1)
    G2 = _sc_gather(T2, src2, NH)
    GB = _sc_gather(B, dst2, EC)

    m3parts = pl.pallas_call(
        _edge2_body_split,
        grid=(EG,),
        in_specs=[_edge_spec(EC), _edge_spec(NH), _edge_spec(EC),
                  _w_spec((EC, EC)), _w_spec((EC,)),
                  _w_spec((EC, EC)), _w_spec((EC,)), _w_spec((EC,)),
                  _w_spec((EC,)), _w_spec((EC, DD)), _w_spec((DD,))],
        out_specs=tuple(_edge_spec(SW) for _ in range(3)),
        out_shape=tuple(jax.ShapeDtypeStruct((E, SW), jnp.float32)
                        for _ in range(3)),
    )(edge_features, G2, GB, W1[2 * EC:], b1, W2, b2, ln_g, ln_b,
      W_nd[NH:], b_nd)

    acc3 = _sc_segsum([m.reshape(ER, IW, SW) for m in m3parts], dst2)
    dens_upd = jnp.concatenate(acc3, axis=-1)[:, :DD] * inv_deg
    dens = density_features + dens_upd.reshape(N, DL, DC)
    dscale = jax.lax.rsqrt(jnp.mean(dens * dens, axis=(1, 2), keepdims=True) + 1e-6)
    return dens * dscale
